# baseline (speedup 1.0000x reference)
import functools

import jax
import jax.numpy as jnp
from jax import lax
from jax.experimental import pallas as pl
from jax.experimental.pallas import tpu as pltpu
from jax.experimental.pallas import tpu_sc as plsc

BATCH = 16384
FACTOR = 64
LANES = 16

_info = plsc.get_sparse_core_info()
_NC, _NS = _info.num_cores, _info.num_subcores
_NW = _NC * _NS
_BPW = BATCH // _NW
_GROUPS = _BPW // LANES


def _body(user_hbm, item_i_hbm, item_j_hbm, eu_hbm, ei_hbm,
          out_i_hbm, out_j_hbm,
          idx_u, idx_i, idx_j, u_rows, vi_rows, vj_rows,
          out_i_v, out_j_v, sem_u, sem_i, sem_j):
    wid = lax.axis_index("s") * _NC + lax.axis_index("c")
    base = wid * _BPW

    pltpu.sync_copy(user_hbm.at[pl.ds(base, _BPW)], idx_u)
    pltpu.sync_copy(item_i_hbm.at[pl.ds(base, _BPW)], idx_i)
    pltpu.sync_copy(item_j_hbm.at[pl.ds(base, _BPW)], idx_j)

    cp_u = pltpu.async_copy(eu_hbm.at[idx_u], u_rows, sem_u)
    cp_i = pltpu.async_copy(ei_hbm.at[idx_i], vi_rows, sem_i)
    cp_j = pltpu.async_copy(ei_hbm.at[idx_j], vj_rows, sem_j)
    cp_u.wait()
    cp_i.wait()
    cp_j.wait()
    lane = lax.iota(jnp.int32, LANES)

    def group(g, _):
        row0 = g * LANES
        rows = row0 + lane
        acc_i = jnp.zeros((LANES,), jnp.float32)
        acc_j = jnp.zeros((LANES,), jnp.float32)
        for c in range(FACTOR):
            col = jnp.full((LANES,), c, jnp.int32)
            uv = plsc.load_gather(u_rows, [rows, col])
            iv = plsc.load_gather(vi_rows, [rows, col])
            jv = plsc.load_gather(vj_rows, [rows, col])
            acc_i = acc_i + uv * iv
            acc_j = acc_j + uv * jv
        out_i_v[pl.ds(row0, LANES)] = acc_i
        out_j_v[pl.ds(row0, LANES)] = acc_j
        return _

    lax.fori_loop(0, _GROUPS, group, None)

    pltpu.sync_copy(out_i_v, out_i_hbm.at[pl.ds(base, _BPW)])
    pltpu.sync_copy(out_j_v, out_j_hbm.at[pl.ds(base, _BPW)])


@jax.jit
def _run(user, item_i, item_j, embed_user, embed_item):
    mesh = plsc.VectorSubcoreMesh(core_axis_name="c", subcore_axis_name="s")
    f = functools.partial(
        pl.kernel,
        mesh=mesh,
        out_type=[
            jax.ShapeDtypeStruct((BATCH,), jnp.float32),
            jax.ShapeDtypeStruct((BATCH,), jnp.float32),
        ],
        scratch_types=[
            pltpu.VMEM((_BPW,), jnp.int32),
            pltpu.VMEM((_BPW,), jnp.int32),
            pltpu.VMEM((_BPW,), jnp.int32),
            pltpu.VMEM((_BPW, FACTOR), jnp.float32),
            pltpu.VMEM((_BPW, FACTOR), jnp.float32),
            pltpu.VMEM((_BPW, FACTOR), jnp.float32),
            pltpu.VMEM((_BPW,), jnp.float32),
            pltpu.VMEM((_BPW,), jnp.float32),
            pltpu.SemaphoreType.DMA,
            pltpu.SemaphoreType.DMA,
            pltpu.SemaphoreType.DMA,
        ],
        compiler_params=pltpu.CompilerParams(
            needs_layout_passes=False, use_tc_tiling_on_sc=False
        ),
    )(_body)
    return tuple(f(user, item_i, item_j, embed_user, embed_item))


def kernel(user, item_i, item_j, embed_user, embed_item):
    return _run(user, item_i, item_j, embed_user, embed_item)


# speedup vs baseline: 1.0052x; 1.0052x over previous
import functools

import jax
import jax.numpy as jnp
from jax import lax
from jax.experimental import pallas as pl
from jax.experimental.pallas import tpu as pltpu
from jax.experimental.pallas import tpu_sc as plsc

BATCH = 16384
FACTOR = 64
LANES = 16

_info = plsc.get_sparse_core_info()
_NC, _NS = _info.num_cores, _info.num_subcores
_NW = _NC * _NS
_BPW = BATCH // _NW
_CHUNK = 256
_NCHUNK = _BPW // _CHUNK
_CGROUPS = _CHUNK // LANES


def _body(user_hbm, item_i_hbm, item_j_hbm, eu_hbm, ei_hbm,
          out_i_hbm, out_j_hbm,
          idx_u, idx_i, idx_j, h_u, h_i, h_j,
          u_rows, vi_rows, vj_rows,
          out_i_v, out_j_v, sem_u, sem_i, sem_j):
    wid = lax.axis_index("s") * _NC + lax.axis_index("c")
    base = wid * _BPW

    pltpu.sync_copy(user_hbm.at[pl.ds(base, _BPW)], idx_u)
    pltpu.sync_copy(item_i_hbm.at[pl.ds(base, _BPW)], idx_i)
    pltpu.sync_copy(item_j_hbm.at[pl.ds(base, _BPW)], idx_j)

    lane = lax.iota(jnp.int32, LANES)

    for chunk in range(_NCHUNK):
        c0 = chunk * _CHUNK

        def halve(g, _):
            s = g * LANES
            h_u[pl.ds(s, LANES)] = idx_u[pl.ds(c0 + s, LANES)] >> 1
            h_i[pl.ds(s, LANES)] = idx_i[pl.ds(c0 + s, LANES)] >> 1
            h_j[pl.ds(s, LANES)] = idx_j[pl.ds(c0 + s, LANES)] >> 1
            return _

        lax.fori_loop(0, _CGROUPS, halve, None)

        cp_u = pltpu.async_copy(eu_hbm.at[h_u], u_rows, sem_u)
        cp_i = pltpu.async_copy(ei_hbm.at[h_i], vi_rows, sem_i)
        cp_j = pltpu.async_copy(ei_hbm.at[h_j], vj_rows, sem_j)
        cp_u.wait()
        cp_i.wait()
        cp_j.wait()

        def group(g, _):
            s = g * LANES
            rows = s + lane
            pu = (idx_u[pl.ds(c0 + s, LANES)] & 1) * FACTOR
            pi = (idx_i[pl.ds(c0 + s, LANES)] & 1) * FACTOR
            pj = (idx_j[pl.ds(c0 + s, LANES)] & 1) * FACTOR
            acc_i = jnp.zeros((LANES,), jnp.float32)
            acc_j = jnp.zeros((LANES,), jnp.float32)
            for c in range(FACTOR):
                uv = plsc.load_gather(u_rows, [rows, pu + c])
                iv = plsc.load_gather(vi_rows, [rows, pi + c])
                jv = plsc.load_gather(vj_rows, [rows, pj + c])
                acc_i = acc_i + uv * iv
                acc_j = acc_j + uv * jv
            out_i_v[pl.ds(c0 + s, LANES)] = acc_i
            out_j_v[pl.ds(c0 + s, LANES)] = acc_j
            return _

        lax.fori_loop(0, _CGROUPS, group, None)

    pltpu.sync_copy(out_i_v, out_i_hbm.at[pl.ds(base, _BPW)])
    pltpu.sync_copy(out_j_v, out_j_hbm.at[pl.ds(base, _BPW)])


@jax.jit
def _run(user, item_i, item_j, embed_user, embed_item):
    nu = embed_user.shape[0]
    ni = embed_item.shape[0]
    eu2 = embed_user.reshape(nu // 2, 2 * FACTOR)
    ei2 = embed_item.reshape(ni // 2, 2 * FACTOR)
    mesh = plsc.VectorSubcoreMesh(core_axis_name="c", subcore_axis_name="s")
    f = functools.partial(
        pl.kernel,
        mesh=mesh,
        out_type=[
            jax.ShapeDtypeStruct((BATCH,), jnp.float32),
            jax.ShapeDtypeStruct((BATCH,), jnp.float32),
        ],
        scratch_types=[
            pltpu.VMEM((_BPW,), jnp.int32),
            pltpu.VMEM((_BPW,), jnp.int32),
            pltpu.VMEM((_BPW,), jnp.int32),
            pltpu.VMEM((_CHUNK,), jnp.int32),
            pltpu.VMEM((_CHUNK,), jnp.int32),
            pltpu.VMEM((_CHUNK,), jnp.int32),
            pltpu.VMEM((_CHUNK, 2 * FACTOR), jnp.float32),
            pltpu.VMEM((_CHUNK, 2 * FACTOR), jnp.float32),
            pltpu.VMEM((_CHUNK, 2 * FACTOR), jnp.float32),
            pltpu.VMEM((_BPW,), jnp.float32),
            pltpu.VMEM((_BPW,), jnp.float32),
            pltpu.SemaphoreType.DMA,
            pltpu.SemaphoreType.DMA,
            pltpu.SemaphoreType.DMA,
        ],
        compiler_params=pltpu.CompilerParams(
            needs_layout_passes=False, use_tc_tiling_on_sc=False
        ),
    )(_body)
    return tuple(f(user, item_i, item_j, eu2, ei2))


def kernel(user, item_i, item_j, embed_user, embed_item):
    return _run(user, item_i, item_j, embed_user, embed_item)


# speedup vs baseline: 2.3929x; 2.3804x over previous
import functools

import jax
import jax.numpy as jnp
from jax import lax
from jax.experimental import pallas as pl
from jax.experimental.pallas import tpu as pltpu
from jax.experimental.pallas import tpu_sc as plsc

BATCH = 16384
FACTOR = 64
LANES = 16
NUM_ROWS = 1000000
BLOCKS = (NUM_ROWS + 127) // 128
NLOCAL = (BLOCKS + 31) // 32
NBUCKET = NLOCAL * 3
HCAP = 16400

_info = plsc.get_sparse_core_info()
_NC, _NS = _info.num_cores, _info.num_subcores
_NW = _NC * _NS

_i32 = jnp.int32


def _splat(x):
    return jnp.full((LANES,), x, _i32)


def _sload(ref, i):
    v = ref[pl.ds((i >> 3) << 3, LANES)]
    lane = lax.iota(_i32, LANES)
    return lax.reduce_max(jnp.where(lane == (i & 7), v, -2147483648), (0,))


def _sstore(ref, i, val):
    lane = lax.iota(_i32, LANES)
    plsc.store_scatter(ref, [_splat(i)], _splat(val), mask=lane == 0)


def _sc_body(user_hbm, item_i_hbm, item_j_hbm, eu_hbm, ei_hbm,
             su_hbm, si_hbm, sj_hbm,
             idx_u, idx_i, idx_j, hits, sorted_h, hist, base, cursor,
             worklist, colA, colB, rowbuf,
             semA, semB, semR):
    wid = lax.axis_index("s") * _NC + lax.axis_index("c")
    lane = lax.iota(_i32, LANES)

    pltpu.sync_copy(user_hbm, idx_u)
    pltpu.sync_copy(item_i_hbm, idx_i)
    pltpu.sync_copy(item_j_hbm, idx_j)

    def scan_arr(idx_ref, arr_id):
        def body(i, off):
            v = idx_ref[pl.ds(i * LANES, LANES)]
            blk = v >> 7
            m = (blk & 31) == wid
            local = blk >> 5
            rloc = v & 127
            pos = i * LANES + lane
            code = (local << 23) | (rloc << 16) | (arr_id << 14) | pos
            cum = plsc.cumsum(jnp.where(m, 1, 0))
            plsc.store_scatter(hits, [off + cum - 1], code, mask=m)
            return off + plsc.all_reduce_population_count(m)
        return body

    off = jnp.zeros((LANES,), _i32)
    off = lax.fori_loop(0, BATCH // LANES, scan_arr(idx_u, 0), off)
    off = lax.fori_loop(0, BATCH // LANES, scan_arr(idx_i, 1), off)
    off = lax.fori_loop(0, BATCH // LANES, scan_arr(idx_j, 2), off)
    n = lax.reduce_max(off, (0,))
    sentinel = (NLOCAL << 23)
    plsc.store_scatter(hits, [off + lane], _splat(sentinel),
                       mask=jnp.ones((LANES,), jnp.bool_))
    nch = (n + LANES - 1) // LANES

    def zero_hist(i, _):
        hist[pl.ds(i * LANES, LANES)] = jnp.zeros((LANES,), _i32)
        return _
    lax.fori_loop(0, (NBUCKET + LANES) // LANES, zero_hist, None)

    def histo(t, _):
        hv = hits[pl.ds(t * LANES, LANES)]
        for k in range(LANES):
            code = hv[k]
            b = (code >> 23) * 3 + ((code >> 14) & 3)
            _sstore(hist, b, _sload(hist, b) + 1)
        return _
    lax.fori_loop(0, nch, histo, None)

    def prefix(i, carry):
        v = hist[pl.ds(i * LANES, LANES)]
        cum = plsc.cumsum(v)
        b = carry + cum - v
        base[pl.ds(i * LANES, LANES)] = b
        cursor[pl.ds(i * LANES, LANES)] = b
        return carry + _splat(lax.reduce_sum(v, (0,)))
    lax.fori_loop(0, (NBUCKET + LANES) // LANES, prefix,
                  jnp.zeros((LANES,), _i32))

    def csort(t, _):
        hv = hits[pl.ds(t * LANES, LANES)]
        for k in range(LANES):
            code = hv[k]
            b = (code >> 23) * 3 + ((code >> 14) & 3)
            o = _sload(cursor, b)
            _sstore(cursor, b, o + 1)
            _sstore(sorted_h, o, code)
        return _
    lax.fori_loop(0, nch, csort, None)

    def wl_user(i, off):
        l = i * LANES + lane
        lc = jnp.minimum(l, NLOCAL - 1)
        cu = plsc.load_gather(hist, [lc * 3])
        m = (cu > 0) & (l < NLOCAL)
        cum = plsc.cumsum(jnp.where(m, 1, 0))
        plsc.store_scatter(worklist, [off + cum - 1], l, mask=m)
        return off + plsc.all_reduce_population_count(m)

    def wl_item(i, off):
        l = i * LANES + lane
        lc = jnp.minimum(l, NLOCAL - 1)
        ci = plsc.load_gather(hist, [lc * 3 + 1]) + plsc.load_gather(hist, [lc * 3 + 2])
        m = (ci > 0) & (l < NLOCAL)
        cum = plsc.cumsum(jnp.where(m, 1, 0))
        plsc.store_scatter(worklist, [off + cum - 1], l | 512, mask=m)
        return off + plsc.all_reduce_population_count(m)

    woff = lax.fori_loop(0, (NLOCAL + LANES - 1) // LANES, wl_user,
                         jnp.zeros((LANES,), _i32))
    woff = lax.fori_loop(0, (NLOCAL + LANES - 1) // LANES, wl_item, woff)
    nw = lax.reduce_max(woff, (0,))

    def fire(e, buf, sem):
        @pl.when(e < nw)
        def _f():
            wcode = _sload(worklist, e)
            l = wcode & 511
            r0 = (l * 32 + wid) * 128

            @pl.when(wcode < 512)
            def _u():
                pltpu.async_copy(
                    eu_hbm.at[pl.ds(0, FACTOR), pl.ds(r0, 128)], buf, sem)

            @pl.when(wcode >= 512)
            def _i():
                pltpu.async_copy(
                    ei_hbm.at[pl.ds(0, FACTOR), pl.ds(r0, 128)], buf, sem)

    def do_bucket(b, stage_hbm, buf, slot):
        bs = _sload(base, b)
        cnt = _sload(hist, b)

        def hit(t, slot):
            code = _sload(sorted_h, bs + t)
            rloc = (code >> 16) & 127
            pos = code & 16383
            ring = slot & 7

            @pl.when(slot >= 8)
            def _w():
                pltpu.make_async_copy(
                    rowbuf.at[0], stage_hbm.at[0], semR).wait()

            for c0 in range(0, FACTOR, LANES):
                v = plsc.load_gather(buf, [c0 + lane, _splat(rloc)])
                rowbuf[ring, pl.ds(c0, LANES)] = v
            pltpu.async_copy(rowbuf.at[ring], stage_hbm.at[pos], semR)
            return slot + 1

        return lax.fori_loop(0, cnt, hit, slot)

    def process(e, buf, sem, slot):
        @pl.when(e < nw)
        def _p():
            pltpu.make_async_copy(
                eu_hbm.at[pl.ds(0, FACTOR), pl.ds(0, 128)], buf, sem).wait()

        wcode = _sload(worklist, jnp.minimum(e, jnp.maximum(nw - 1, 0)))
        l = wcode & 511

        def if_user(slot):
            return do_bucket(l * 3, su_hbm, buf, slot)

        def if_item(slot):
            slot = do_bucket(l * 3 + 1, si_hbm, buf, slot)
            return do_bucket(l * 3 + 2, sj_hbm, buf, slot)

        def if_skip(slot):
            return slot

        slot = lax.cond(e < nw,
                        lambda s: lax.cond(wcode < 512, if_user, if_item, s),
                        if_skip, slot)
        return slot

    fire(jnp.int32(0), colA, semA)
    fire(jnp.int32(1), colB, semB)

    def sweep(p, slot):
        e0 = 2 * p
        slot = process(e0, colA, semA, slot)
        fire(e0 + 2, colA, semA)
        slot = process(e0 + 1, colB, semB, slot)
        fire(e0 + 3, colB, semB)
        return slot

    slot = lax.fori_loop(0, (nw + 1) // 2, sweep, jnp.int32(0))

    def drain(k, _):
        @pl.when(k < jnp.minimum(slot, 8))
        def _d():
            pltpu.make_async_copy(rowbuf.at[0], su_hbm.at[0], semR).wait()
        return _
    lax.fori_loop(0, 8, drain, None)


def _tc_body(su_ref, si_ref, sj_ref, oi_ref, oj_ref):
    u = su_ref[:, :FACTOR]
    vi = si_ref[:, :FACTOR]
    vj = sj_ref[:, :FACTOR]
    oi_ref[...] = jnp.sum(u * vi, axis=1)
    oj_ref[...] = jnp.sum(u * vj, axis=1)


@jax.jit
def _run(user, item_i, item_j, embed_user, embed_item):
    eu_t = embed_user.T
    ei_t = embed_item.T
    mesh = plsc.VectorSubcoreMesh(core_axis_name="c", subcore_axis_name="s")
    phase1 = functools.partial(
        pl.kernel,
        mesh=mesh,
        out_type=[
            jax.ShapeDtypeStruct((BATCH, 128), jnp.float32),
            jax.ShapeDtypeStruct((BATCH, 128), jnp.float32),
            jax.ShapeDtypeStruct((BATCH, 128), jnp.float32),
        ],
        scratch_types=[
            pltpu.VMEM((BATCH,), _i32),
            pltpu.VMEM((BATCH,), _i32),
            pltpu.VMEM((BATCH,), _i32),
            pltpu.VMEM((HCAP,), _i32),
            pltpu.VMEM((HCAP,), _i32),
            pltpu.VMEM((NBUCKET + LANES,), _i32),
            pltpu.VMEM((NBUCKET + LANES,), _i32),
            pltpu.VMEM((NBUCKET + LANES,), _i32),
            pltpu.VMEM((2 * NLOCAL + LANES,), _i32),
            pltpu.VMEM((FACTOR, 128), jnp.float32),
            pltpu.VMEM((FACTOR, 128), jnp.float32),
            pltpu.VMEM((8, 128), jnp.float32),
            pltpu.SemaphoreType.DMA,
            pltpu.SemaphoreType.DMA,
            pltpu.SemaphoreType.DMA,
        ],
        compiler_params=pltpu.CompilerParams(
            needs_layout_passes=False, use_tc_tiling_on_sc=True
        ),
    )(_sc_body)
    su, si, sj = phase1(user, item_i, item_j, eu_t, ei_t)

    grid = 16
    rows = BATCH // grid
    oi, oj = pl.pallas_call(
        _tc_body,
        grid=(grid,),
        in_specs=[
            pl.BlockSpec((rows, 128), lambda i: (i, 0)),
            pl.BlockSpec((rows, 128), lambda i: (i, 0)),
            pl.BlockSpec((rows, 128), lambda i: (i, 0)),
        ],
        out_specs=[
            pl.BlockSpec((rows,), lambda i: (i,)),
            pl.BlockSpec((rows,), lambda i: (i,)),
        ],
        out_shape=[
            jax.ShapeDtypeStruct((BATCH,), jnp.float32),
            jax.ShapeDtypeStruct((BATCH,), jnp.float32),
        ],
    )(su, si, sj)
    return (oi, oj)


def kernel(user, item_i, item_j, embed_user, embed_item):
    return _run(user, item_i, item_j, embed_user, embed_item)


# speedup vs baseline: 2.9064x; 1.2146x over previous
import functools

import jax
import jax.numpy as jnp
from jax import lax
from jax.experimental import pallas as pl
from jax.experimental.pallas import tpu as pltpu
from jax.experimental.pallas import tpu_sc as plsc

BATCH = 16384
FACTOR = 64
LANES = 16
NUM_ROWS = 1000000
BLOCKS = (NUM_ROWS + 127) // 128
WBLK = 4
WCOLS = WBLK * 128
NWIN = 62
NBUCKET = NWIN * 3
HCAP = 4096

_info = plsc.get_sparse_core_info()
_NC, _NS = _info.num_cores, _info.num_subcores
_NW = _NC * _NS

_i32 = jnp.int32


def _splat(x):
    return jnp.full((LANES,), x, _i32)


def _sload(ref, i):
    v = ref[pl.ds((i >> 3) << 3, LANES)]
    g = v[_splat(i & 7)]
    return g[0]


def _sstore(ref, i, val):
    lane = lax.iota(_i32, LANES)
    plsc.store_scatter(ref, [_splat(i)], _splat(val), mask=lane == 0)


def _sc_body(user_hbm, item_i_hbm, item_j_hbm, eu_hbm, ei_hbm,
             su_hbm, si_hbm, sj_hbm,
             idx_u, idx_i, idx_j, hits, sorted_h, hist, base, cursor,
             colA, colB, rowbuf,
             semA, semB, semR):
    wid = lax.axis_index("s") * _NC + lax.axis_index("c")
    lane = lax.iota(_i32, LANES)
    lo = (wid * BLOCKS) >> 5
    hi = ((wid + 1) * BLOCKS) >> 5

    pltpu.sync_copy(user_hbm, idx_u)
    pltpu.sync_copy(item_i_hbm, idx_i)
    pltpu.sync_copy(item_j_hbm, idx_j)

    def scan_arr(idx_ref, arr_id):
        def body(i, off):
            v = idx_ref[pl.ds(i * LANES, LANES)]
            blk = v >> 7
            m = (blk >= lo) & (blk < hi)
            pc = plsc.all_reduce_population_count(m)

            @pl.when(pc[0] > 0)
            def _hit():
                local = blk - lo
                rloc = v & 127
                pos = i * LANES + lane
                code = (local << 23) | (rloc << 16) | (arr_id << 14) | pos
                cum = plsc.cumsum(jnp.where(m, 1, 0))
                plsc.store_scatter(hits, [off + cum - 1], code, mask=m)

            return off + pc
        return body

    off = jnp.zeros((LANES,), _i32)
    off = lax.fori_loop(0, BATCH // LANES, scan_arr(idx_u, 0), off)
    off = lax.fori_loop(0, BATCH // LANES, scan_arr(idx_i, 1), off)
    off = lax.fori_loop(0, BATCH // LANES, scan_arr(idx_j, 2), off)
    n = off[0]
    sentinel = (NWIN * WBLK) << 23
    plsc.store_scatter(hits, [off + lane], _splat(sentinel),
                       mask=jnp.ones((LANES,), jnp.bool_))
    nch = (n + LANES - 1) // LANES

    def zero_hist(i, _):
        hist[pl.ds(i * LANES, LANES)] = jnp.zeros((LANES,), _i32)
        return _
    lax.fori_loop(0, (NBUCKET + LANES) // LANES, zero_hist, None)

    def histo(t, _):
        hv = hits[pl.ds(t * LANES, LANES)]
        bv = ((hv >> 23) >> 2) * 3 + ((hv >> 14) & 3)
        for k in range(LANES):
            b = bv[k]
            _sstore(hist, b, _sload(hist, b) + 1)
        return _
    lax.fori_loop(0, nch, histo, None)

    def prefix(i, carry):
        v = hist[pl.ds(i * LANES, LANES)]
        cum = plsc.cumsum(v)
        b = carry + cum - v
        base[pl.ds(i * LANES, LANES)] = b
        cursor[pl.ds(i * LANES, LANES)] = b
        return carry + _splat(lax.reduce_sum(v, (0,)))
    lax.fori_loop(0, (NBUCKET + LANES) // LANES, prefix,
                  jnp.zeros((LANES,), _i32))

    def csort(t, _):
        hv = hits[pl.ds(t * LANES, LANES)]
        bv = ((hv >> 23) >> 2) * 3 + ((hv >> 14) & 3)
        for k in range(LANES):
            b = bv[k]
            o = _sload(cursor, b)
            _sstore(cursor, b, o + 1)
            _sstore(sorted_h, o, hv[k])
        return _
    lax.fori_loop(0, nch, csort, None)

    def wbase_of(w):
        uw = jnp.where(w < NWIN, w, w - NWIN)
        return jnp.minimum(lo + uw * WBLK, BLOCKS - WBLK)

    def fire(e, buf, sem):
        @pl.when(e < 2 * NWIN)
        def _f():
            r0 = wbase_of(e) * 128

            @pl.when(e < NWIN)
            def _u():
                pltpu.async_copy(
                    eu_hbm.at[pl.ds(0, FACTOR), pl.ds(r0, WCOLS)], buf, sem)

            @pl.when(e >= NWIN)
            def _i():
                pltpu.async_copy(
                    ei_hbm.at[pl.ds(0, FACTOR), pl.ds(r0, WCOLS)], buf, sem)

    def do_bucket(b, shift, stage_hbm, buf, slot):
        bs = _sload(base, b)
        cnt = _sload(hist, b)

        def hit(t, slot):
            code = _sload(sorted_h, bs + t)
            col = shift + (((code >> 23) & 3) << 7) + ((code >> 16) & 127)
            pos = code & 16383
            ring = slot & 7

            @pl.when(slot >= 8)
            def _w():
                pltpu.make_async_copy(
                    rowbuf.at[0], stage_hbm.at[0], semR).wait()

            for c0 in range(0, FACTOR, LANES):
                v = plsc.load_gather(buf, [c0 + lane, _splat(col)])
                rowbuf[ring, pl.ds(c0, LANES)] = v
            pltpu.async_copy(rowbuf.at[ring], stage_hbm.at[pos], semR)
            return slot + 1

        return lax.fori_loop(0, cnt, hit, slot)

    def process(e, buf, sem, slot):
        pltpu.make_async_copy(
            eu_hbm.at[pl.ds(0, FACTOR), pl.ds(0, WCOLS)], buf, sem).wait()
        uw = jnp.where(e < NWIN, e, e - NWIN)
        shift = ((lo + uw * WBLK) - wbase_of(e)) * 128

        def if_user(slot):
            return do_bucket(uw * 3, shift, su_hbm, buf, slot)

        def if_item(slot):
            slot = do_bucket(uw * 3 + 1, shift, si_hbm, buf, slot)
            return do_bucket(uw * 3 + 2, shift, sj_hbm, buf, slot)

        return lax.cond(e < NWIN, if_user, if_item, slot)

    fire(jnp.int32(0), colA, semA)
    fire(jnp.int32(1), colB, semB)

    def sweep(p, slot):
        e0 = 2 * p
        slot = process(e0, colA, semA, slot)
        fire(e0 + 2, colA, semA)
        slot = process(e0 + 1, colB, semB, slot)
        fire(e0 + 3, colB, semB)
        return slot

    slot = lax.fori_loop(0, NWIN, sweep, jnp.int32(0))

    def drain(k, _):
        @pl.when(k < jnp.minimum(slot, 8))
        def _d():
            pltpu.make_async_copy(rowbuf.at[0], su_hbm.at[0], semR).wait()
        return _
    lax.fori_loop(0, 8, drain, None)


def _tc_body(su_ref, si_ref, sj_ref, oi_ref, oj_ref):
    u = su_ref[:, :FACTOR]
    vi = si_ref[:, :FACTOR]
    vj = sj_ref[:, :FACTOR]
    oi_ref[...] = jnp.sum(u * vi, axis=1)
    oj_ref[...] = jnp.sum(u * vj, axis=1)


@jax.jit
def _run(user, item_i, item_j, embed_user, embed_item):
    eu_t = embed_user.T
    ei_t = embed_item.T
    mesh = plsc.VectorSubcoreMesh(core_axis_name="c", subcore_axis_name="s")
    phase1 = functools.partial(
        pl.kernel,
        mesh=mesh,
        out_type=[
            jax.ShapeDtypeStruct((BATCH, 128), jnp.float32),
            jax.ShapeDtypeStruct((BATCH, 128), jnp.float32),
            jax.ShapeDtypeStruct((BATCH, 128), jnp.float32),
        ],
        scratch_types=[
            pltpu.VMEM((BATCH,), _i32),
            pltpu.VMEM((BATCH,), _i32),
            pltpu.VMEM((BATCH,), _i32),
            pltpu.VMEM((HCAP,), _i32),
            pltpu.VMEM((HCAP,), _i32),
            pltpu.VMEM((NBUCKET + LANES,), _i32),
            pltpu.VMEM((NBUCKET + LANES,), _i32),
            pltpu.VMEM((NBUCKET + LANES,), _i32),
            pltpu.VMEM((FACTOR, WCOLS), jnp.float32),
            pltpu.VMEM((FACTOR, WCOLS), jnp.float32),
            pltpu.VMEM((8, 128), jnp.float32),
            pltpu.SemaphoreType.DMA,
            pltpu.SemaphoreType.DMA,
            pltpu.SemaphoreType.DMA,
        ],
        compiler_params=pltpu.CompilerParams(
            needs_layout_passes=False, use_tc_tiling_on_sc=True
        ),
    )(_sc_body)
    su, si, sj = phase1(user, item_i, item_j, eu_t, ei_t)

    grid = 16
    rows = BATCH // grid
    oi, oj = pl.pallas_call(
        _tc_body,
        grid=(grid,),
        in_specs=[
            pl.BlockSpec((rows, 128), lambda i: (i, 0)),
            pl.BlockSpec((rows, 128), lambda i: (i, 0)),
            pl.BlockSpec((rows, 128), lambda i: (i, 0)),
        ],
        out_specs=[
            pl.BlockSpec((rows,), lambda i: (i,)),
            pl.BlockSpec((rows,), lambda i: (i,)),
        ],
        out_shape=[
            jax.ShapeDtypeStruct((BATCH,), jnp.float32),
            jax.ShapeDtypeStruct((BATCH,), jnp.float32),
        ],
    )(su, si, sj)
    return (oi, oj)


def kernel(user, item_i, item_j, embed_user, embed_item):
    return _run(user, item_i, item_j, embed_user, embed_item)


# speedup vs baseline: 3.1312x; 1.0773x over previous
import functools

import jax
import jax.numpy as jnp
from jax import lax
from jax.experimental import pallas as pl
from jax.experimental.pallas import tpu as pltpu
from jax.experimental.pallas import tpu_sc as plsc

BATCH = 16384
FACTOR = 64
LANES = 16
NUM_ROWS = 1000000
BLOCKS = (NUM_ROWS + 127) // 128
WBLK = 4
WCOLS = WBLK * 128
NWIN = 62
NBUCKET = NWIN * 3
HCAP = 4096

_info = plsc.get_sparse_core_info()
_NC, _NS = _info.num_cores, _info.num_subcores
_NW = _NC * _NS

_i32 = jnp.int32


def _splat(x):
    return jnp.full((LANES,), x, _i32)


def _sload(ref, i):
    v = ref[pl.ds((i >> 3) << 3, LANES)]
    g = v[_splat(i & 7)]
    return g[0]


def _sstore(ref, i, val):
    lane = lax.iota(_i32, LANES)
    plsc.store_scatter(ref, [_splat(i)], _splat(val), mask=lane == 0)


ICHUNK = 4096


def _sc_body(user_hbm, item_i_hbm, item_j_hbm, eu_hbm, ei_hbm,
             su_hbm, si_hbm, sj_hbm,
             idxA, idxB, hits, sorted_h, hist, base, cursor,
             colA, colB, colC, rowbuf,
             semA, semB, semC, semR):
    wid = lax.axis_index("s") * _NC + lax.axis_index("c")
    lane = lax.iota(_i32, LANES)
    lo = (wid * BLOCKS) >> 5
    hi = ((wid + 1) * BLOCKS) >> 5

    idx_refs = (user_hbm, item_i_hbm, item_j_hbm)
    pieces = [(a, k) for a in range(3) for k in range(BATCH // ICHUNK)]
    ibufs = (idxA, idxB)
    isems = (semA, semB)

    def ifire(p):
        a, k = pieces[p]
        return pltpu.async_copy(
            idx_refs[a].at[pl.ds(k * ICHUNK, ICHUNK)], ibufs[p % 2],
            isems[p % 2])

    def scan_piece(buf, arr_id, k):
        def body(i, off):
            v = buf[pl.ds(i * LANES, LANES)]
            blk = v >> 7
            m = (blk >= lo) & (blk < hi)
            pc = plsc.all_reduce_population_count(m)

            @pl.when(pc[0] > 0)
            def _hit():
                local = blk - lo
                rloc = v & 127
                pos = k * ICHUNK + i * LANES + lane
                code = (local << 23) | (rloc << 16) | (arr_id << 14) | pos
                cum = plsc.cumsum(jnp.where(m, 1, 0))
                plsc.store_scatter(hits, [off + cum - 1], code, mask=m)

            return off + pc
        return body

    off = jnp.zeros((LANES,), _i32)
    cps = {0: ifire(0)}
    for p in range(len(pieces)):
        if p + 1 < len(pieces):
            cps[p + 1] = ifire(p + 1)
        cps.pop(p).wait()
        a, k = pieces[p]
        off = lax.fori_loop(0, ICHUNK // LANES, scan_piece(ibufs[p % 2], a, k),
                            off)
    n = off[0]
    sentinel = (NWIN * WBLK) << 23
    plsc.store_scatter(hits, [off + lane], _splat(sentinel),
                       mask=jnp.ones((LANES,), jnp.bool_))
    nch = (n + LANES - 1) // LANES

    def zero_hist(i, _):
        hist[pl.ds(i * LANES, LANES)] = jnp.zeros((LANES,), _i32)
        return _
    lax.fori_loop(0, (NBUCKET + LANES) // LANES, zero_hist, None)

    def histo(t, _):
        hv = hits[pl.ds(t * LANES, LANES)]
        bv = ((hv >> 23) >> 2) * 3 + ((hv >> 14) & 3)
        for k in range(LANES):
            b = bv[k]
            _sstore(hist, b, _sload(hist, b) + 1)
        return _
    lax.fori_loop(0, nch, histo, None)

    def prefix(i, carry):
        v = hist[pl.ds(i * LANES, LANES)]
        cum = plsc.cumsum(v)
        b = carry + cum - v
        base[pl.ds(i * LANES, LANES)] = b
        cursor[pl.ds(i * LANES, LANES)] = b
        return carry + _splat(lax.reduce_sum(v, (0,)))
    lax.fori_loop(0, (NBUCKET + LANES) // LANES, prefix,
                  jnp.zeros((LANES,), _i32))

    def csort(t, _):
        hv = hits[pl.ds(t * LANES, LANES)]
        bv = ((hv >> 23) >> 2) * 3 + ((hv >> 14) & 3)
        for k in range(LANES):
            b = bv[k]
            o = _sload(cursor, b)
            _sstore(cursor, b, o + 1)
            _sstore(sorted_h, o, hv[k])
        return _
    lax.fori_loop(0, nch, csort, None)

    def wbase_of(w):
        uw = jnp.where(w < NWIN, w, w - NWIN)
        return jnp.minimum(lo + uw * WBLK, BLOCKS - WBLK)

    def fire(e, buf, sem):
        @pl.when(e < 2 * NWIN)
        def _f():
            r0 = wbase_of(e) * 128

            @pl.when(e < NWIN)
            def _u():
                pltpu.async_copy(
                    eu_hbm.at[pl.ds(0, FACTOR), pl.ds(r0, WCOLS)], buf, sem)

            @pl.when(e >= NWIN)
            def _i():
                pltpu.async_copy(
                    ei_hbm.at[pl.ds(0, FACTOR), pl.ds(r0, WCOLS)], buf, sem)

    def do_bucket(b, shift, stage_hbm, buf, slot):
        bs = _sload(base, b)
        cnt = _sload(hist, b)

        def hit(t, slot):
            code = _sload(sorted_h, bs + t)
            col = shift + (((code >> 23) & 3) << 7) + ((code >> 16) & 127)
            pos = code & 16383
            ring = slot & 7

            @pl.when(slot >= 8)
            def _w():
                pltpu.make_async_copy(
                    rowbuf.at[0], stage_hbm.at[0], semR).wait()

            for c0 in range(0, FACTOR, LANES):
                v = plsc.load_gather(buf, [c0 + lane, _splat(col)])
                rowbuf[ring, pl.ds(c0, LANES)] = v
            pltpu.async_copy(rowbuf.at[ring], stage_hbm.at[pos], semR)
            return slot + 1

        return lax.fori_loop(0, cnt, hit, slot)

    def process(e, buf, sem, slot):
        def live(slot):
            pltpu.make_async_copy(
                eu_hbm.at[pl.ds(0, FACTOR), pl.ds(0, WCOLS)], buf, sem).wait()
            uw = jnp.where(e < NWIN, e, e - NWIN)
            shift = ((lo + uw * WBLK) - wbase_of(e)) * 128

            def if_user(slot):
                return do_bucket(uw * 3, shift, su_hbm, buf, slot)

            def if_item(slot):
                slot = do_bucket(uw * 3 + 1, shift, si_hbm, buf, slot)
                return do_bucket(uw * 3 + 2, shift, sj_hbm, buf, slot)

            return lax.cond(e < NWIN, if_user, if_item, slot)

        return lax.cond(e < 2 * NWIN, live, lambda s: s, slot)

    fire(jnp.int32(0), colA, semA)
    fire(jnp.int32(1), colB, semB)
    fire(jnp.int32(2), colC, semC)

    def sweep(p, slot):
        e0 = 3 * p
        slot = process(e0, colA, semA, slot)
        fire(e0 + 3, colA, semA)
        slot = process(e0 + 1, colB, semB, slot)
        fire(e0 + 4, colB, semB)
        slot = process(e0 + 2, colC, semC, slot)
        fire(e0 + 5, colC, semC)
        return slot

    slot = lax.fori_loop(0, (2 * NWIN + 2) // 3, sweep, jnp.int32(0))

    def drain(k, _):
        @pl.when(k < jnp.minimum(slot, 8))
        def _d():
            pltpu.make_async_copy(rowbuf.at[0], su_hbm.at[0], semR).wait()
        return _
    lax.fori_loop(0, 8, drain, None)


def _tc_body(su_ref, si_ref, sj_ref, oi_ref, oj_ref):
    u = su_ref[:, :FACTOR]
    vi = si_ref[:, :FACTOR]
    vj = sj_ref[:, :FACTOR]
    oi_ref[...] = jnp.sum(u * vi, axis=1)
    oj_ref[...] = jnp.sum(u * vj, axis=1)


@jax.jit
def _run(user, item_i, item_j, embed_user, embed_item):
    eu_t = embed_user.T
    ei_t = embed_item.T
    mesh = plsc.VectorSubcoreMesh(core_axis_name="c", subcore_axis_name="s")
    phase1 = functools.partial(
        pl.kernel,
        mesh=mesh,
        out_type=[
            jax.ShapeDtypeStruct((BATCH, 128), jnp.float32),
            jax.ShapeDtypeStruct((BATCH, 128), jnp.float32),
            jax.ShapeDtypeStruct((BATCH, 128), jnp.float32),
        ],
        scratch_types=[
            pltpu.VMEM((ICHUNK,), _i32),
            pltpu.VMEM((ICHUNK,), _i32),
            pltpu.VMEM((HCAP,), _i32),
            pltpu.VMEM((HCAP,), _i32),
            pltpu.VMEM((NBUCKET + LANES,), _i32),
            pltpu.VMEM((NBUCKET + LANES,), _i32),
            pltpu.VMEM((NBUCKET + LANES,), _i32),
            pltpu.VMEM((FACTOR, WCOLS), jnp.float32),
            pltpu.VMEM((FACTOR, WCOLS), jnp.float32),
            pltpu.VMEM((FACTOR, WCOLS), jnp.float32),
            pltpu.VMEM((8, 128), jnp.float32),
            pltpu.SemaphoreType.DMA,
            pltpu.SemaphoreType.DMA,
            pltpu.SemaphoreType.DMA,
            pltpu.SemaphoreType.DMA,
        ],
        compiler_params=pltpu.CompilerParams(
            needs_layout_passes=False, use_tc_tiling_on_sc=True
        ),
    )(_sc_body)
    su, si, sj = phase1(user, item_i, item_j, eu_t, ei_t)

    grid = 16
    rows = BATCH // grid
    oi, oj = pl.pallas_call(
        _tc_body,
        grid=(grid,),
        in_specs=[
            pl.BlockSpec((rows, 128), lambda i: (i, 0)),
            pl.BlockSpec((rows, 128), lambda i: (i, 0)),
            pl.BlockSpec((rows, 128), lambda i: (i, 0)),
        ],
        out_specs=[
            pl.BlockSpec((rows,), lambda i: (i,)),
            pl.BlockSpec((rows,), lambda i: (i,)),
        ],
        out_shape=[
            jax.ShapeDtypeStruct((BATCH,), jnp.float32),
            jax.ShapeDtypeStruct((BATCH,), jnp.float32),
        ],
    )(su, si, sj)
    return (oi, oj)


def kernel(user, item_i, item_j, embed_user, embed_item):
    return _run(user, item_i, item_j, embed_user, embed_item)


# speedup vs baseline: 3.4786x; 1.1109x over previous
import functools

import jax
import jax.numpy as jnp
from jax import lax
from jax.experimental import pallas as pl
from jax.experimental.pallas import tpu as pltpu
from jax.experimental.pallas import tpu_sc as plsc

BATCH = 16384
FACTOR = 64
LANES = 16
NUM_ROWS = 1000000
BLOCKS = (NUM_ROWS + 127) // 128
WBLK = 4
WCOLS = WBLK * 128
NWIN = 62
NBUCKET = NWIN * 3
HCAP = 4096

_info = plsc.get_sparse_core_info()
_NC, _NS = _info.num_cores, _info.num_subcores
_NW = _NC * _NS

_i32 = jnp.int32


def _splat(x):
    return jnp.full((LANES,), x, _i32)


def _sload(ref, i):
    v = ref[pl.ds((i >> 3) << 3, LANES)]
    g = v[_splat(i & 7)]
    return g[0]


def _sstore(ref, i, val):
    lane = lax.iota(_i32, LANES)
    plsc.store_scatter(ref, [_splat(i)], _splat(val), mask=lane == 0)


ICHUNK = 4096


def _sc_body(user_hbm, item_i_hbm, item_j_hbm, eu_hbm, ei_hbm,
             su_hbm, si_hbm, sj_hbm,
             idxA, idxB, hits, sorted_h, hist, base, cursor,
             colA, colB, colC, rowbuf,
             semA, semB, semC, semR):
    wid = lax.axis_index("s") * _NC + lax.axis_index("c")
    lane = lax.iota(_i32, LANES)
    lo = (wid * BLOCKS) >> 5
    hi = ((wid + 1) * BLOCKS) >> 5

    def wbase_of(w):
        uw = jnp.where(w < NWIN, w, w - NWIN)
        return jnp.minimum(lo + uw * WBLK, BLOCKS - WBLK)

    def fire(e, buf, sem):
        @pl.when(e < 2 * NWIN)
        def _f():
            r0 = wbase_of(e) * 128

            @pl.when(e < NWIN)
            def _u():
                pltpu.async_copy(
                    eu_hbm.at[pl.ds(0, FACTOR), pl.ds(r0, WCOLS)], buf, sem)

            @pl.when(e >= NWIN)
            def _i():
                pltpu.async_copy(
                    ei_hbm.at[pl.ds(0, FACTOR), pl.ds(r0, WCOLS)], buf, sem)

    fire(jnp.int32(0), colA, semA)
    fire(jnp.int32(1), colB, semB)

    idx_refs = (user_hbm, item_i_hbm, item_j_hbm)
    pieces = [(a, k) for a in range(3) for k in range(BATCH // ICHUNK)]
    ibufs = (idxA, idxB)
    isems = (semC, semR)

    def ifire(p):
        a, k = pieces[p]
        return pltpu.async_copy(
            idx_refs[a].at[pl.ds(k * ICHUNK, ICHUNK)], ibufs[p % 2],
            isems[p % 2])

    def scan_piece(buf, arr_id, k):
        def body(i, off):
            v = buf[pl.ds(i * LANES, LANES)]
            blk = v >> 7
            m = (blk >= lo) & (blk < hi)
            pc = plsc.all_reduce_population_count(m)

            @pl.when(pc[0] > 0)
            def _hit():
                local = blk - lo
                rloc = v & 127
                pos = k * ICHUNK + i * LANES + lane
                code = (local << 23) | (rloc << 16) | (arr_id << 14) | pos
                cum = plsc.cumsum(jnp.where(m, 1, 0))
                plsc.store_scatter(hits, [off + cum - 1], code, mask=m)

            return off + pc
        return body

    off = jnp.zeros((LANES,), _i32)
    cps = {0: ifire(0)}
    for p in range(len(pieces)):
        if p + 1 < len(pieces):
            cps[p + 1] = ifire(p + 1)
        cps.pop(p).wait()
        a, k = pieces[p]
        off = lax.fori_loop(0, ICHUNK // LANES, scan_piece(ibufs[p % 2], a, k),
                            off)
    n = off[0]
    sentinel = (NWIN * WBLK) << 23
    plsc.store_scatter(hits, [off + lane], _splat(sentinel),
                       mask=jnp.ones((LANES,), jnp.bool_))
    nch = (n + LANES - 1) // LANES

    def zero_hist(i, _):
        hist[pl.ds(i * LANES, LANES)] = jnp.zeros((LANES,), _i32)
        return _
    lax.fori_loop(0, (NBUCKET + LANES) // LANES, zero_hist, None)

    lane0 = lane == 0

    def histo(t, _):
        hv = hits[pl.ds(t * LANES, LANES)]
        bv = ((hv >> 23) >> 2) * 3 + ((hv >> 14) & 3)
        for k in range(LANES):
            bk = bv[_splat(k)]
            c = plsc.load_gather(hist, [bk])
            plsc.store_scatter(hist, [bk], c + 1, mask=lane0)
        return _
    lax.fori_loop(0, nch, histo, None)

    def prefix(i, carry):
        v = hist[pl.ds(i * LANES, LANES)]
        cum = plsc.cumsum(v)
        b = carry + cum - v
        base[pl.ds(i * LANES, LANES)] = b
        cursor[pl.ds(i * LANES, LANES)] = b
        return carry + _splat(lax.reduce_sum(v, (0,)))
    lax.fori_loop(0, (NBUCKET + LANES) // LANES, prefix,
                  jnp.zeros((LANES,), _i32))

    def csort(t, _):
        hv = hits[pl.ds(t * LANES, LANES)]
        bv = ((hv >> 23) >> 2) * 3 + ((hv >> 14) & 3)
        for k in range(LANES):
            bk = bv[_splat(k)]
            o = plsc.load_gather(cursor, [bk])
            plsc.store_scatter(cursor, [bk], o + 1, mask=lane0)
            plsc.store_scatter(sorted_h, [o], hv[_splat(k)], mask=lane0)
        return _
    lax.fori_loop(0, nch, csort, None)

    def do_bucket(b, shift, stage_hbm, buf, slot):
        bs = _sload(base, b)
        cnt = _sload(hist, b)

        def hit(t, slot):
            code = _sload(sorted_h, bs + t)
            col = shift + (((code >> 23) & 3) << 7) + ((code >> 16) & 127)
            pos = code & 16383
            ring = slot & 7

            @pl.when(slot >= 8)
            def _w():
                pltpu.make_async_copy(
                    rowbuf.at[0], stage_hbm.at[0], semR).wait()

            for c0 in range(0, FACTOR, LANES):
                v = plsc.load_gather(buf, [c0 + lane, _splat(col)])
                rowbuf[ring, pl.ds(c0, LANES)] = v
            pltpu.async_copy(rowbuf.at[ring], stage_hbm.at[pos], semR)
            return slot + 1

        return lax.fori_loop(0, cnt, hit, slot)

    def process(e, buf, sem, slot):
        def live(slot):
            pltpu.make_async_copy(
                eu_hbm.at[pl.ds(0, FACTOR), pl.ds(0, WCOLS)], buf, sem).wait()
            uw = jnp.where(e < NWIN, e, e - NWIN)
            shift = ((lo + uw * WBLK) - wbase_of(e)) * 128

            def if_user(slot):
                return do_bucket(uw * 3, shift, su_hbm, buf, slot)

            def if_item(slot):
                slot = do_bucket(uw * 3 + 1, shift, si_hbm, buf, slot)
                return do_bucket(uw * 3 + 2, shift, sj_hbm, buf, slot)

            return lax.cond(e < NWIN, if_user, if_item, slot)

        return lax.cond(e < 2 * NWIN, live, lambda s: s, slot)

    fire(jnp.int32(2), colC, semC)

    def sweep(p, slot):
        e0 = 3 * p
        slot = process(e0, colA, semA, slot)
        fire(e0 + 3, colA, semA)
        slot = process(e0 + 1, colB, semB, slot)
        fire(e0 + 4, colB, semB)
        slot = process(e0 + 2, colC, semC, slot)
        fire(e0 + 5, colC, semC)
        return slot

    slot = lax.fori_loop(0, (2 * NWIN + 2) // 3, sweep, jnp.int32(0))

    def drain(k, _):
        @pl.when(k < jnp.minimum(slot, 8))
        def _d():
            pltpu.make_async_copy(rowbuf.at[0], su_hbm.at[0], semR).wait()
        return _
    lax.fori_loop(0, 8, drain, None)


def _tc_body(su_ref, si_ref, sj_ref, oi_ref, oj_ref):
    u = su_ref[:, :FACTOR]
    vi = si_ref[:, :FACTOR]
    vj = sj_ref[:, :FACTOR]
    oi_ref[...] = jnp.sum(u * vi, axis=1)
    oj_ref[...] = jnp.sum(u * vj, axis=1)


@jax.jit
def _run(user, item_i, item_j, embed_user, embed_item):
    eu_t = embed_user.T
    ei_t = embed_item.T
    mesh = plsc.VectorSubcoreMesh(core_axis_name="c", subcore_axis_name="s")
    phase1 = functools.partial(
        pl.kernel,
        mesh=mesh,
        out_type=[
            jax.ShapeDtypeStruct((BATCH, 128), jnp.float32),
            jax.ShapeDtypeStruct((BATCH, 128), jnp.float32),
            jax.ShapeDtypeStruct((BATCH, 128), jnp.float32),
        ],
        scratch_types=[
            pltpu.VMEM((ICHUNK,), _i32),
            pltpu.VMEM((ICHUNK,), _i32),
            pltpu.VMEM((HCAP,), _i32),
            pltpu.VMEM((HCAP,), _i32),
            pltpu.VMEM((NBUCKET + LANES,), _i32),
            pltpu.VMEM((NBUCKET + LANES,), _i32),
            pltpu.VMEM((NBUCKET + LANES,), _i32),
            pltpu.VMEM((FACTOR, WCOLS), jnp.float32),
            pltpu.VMEM((FACTOR, WCOLS), jnp.float32),
            pltpu.VMEM((FACTOR, WCOLS), jnp.float32),
            pltpu.VMEM((8, 128), jnp.float32),
            pltpu.SemaphoreType.DMA,
            pltpu.SemaphoreType.DMA,
            pltpu.SemaphoreType.DMA,
            pltpu.SemaphoreType.DMA,
        ],
        compiler_params=pltpu.CompilerParams(
            needs_layout_passes=False, use_tc_tiling_on_sc=True
        ),
    )(_sc_body)
    su, si, sj = phase1(user, item_i, item_j, eu_t, ei_t)

    grid = 16
    rows = BATCH // grid
    oi, oj = pl.pallas_call(
        _tc_body,
        grid=(grid,),
        in_specs=[
            pl.BlockSpec((rows, 128), lambda i: (i, 0)),
            pl.BlockSpec((rows, 128), lambda i: (i, 0)),
            pl.BlockSpec((rows, 128), lambda i: (i, 0)),
        ],
        out_specs=[
            pl.BlockSpec((rows,), lambda i: (i,)),
            pl.BlockSpec((rows,), lambda i: (i,)),
        ],
        out_shape=[
            jax.ShapeDtypeStruct((BATCH,), jnp.float32),
            jax.ShapeDtypeStruct((BATCH,), jnp.float32),
        ],
    )(su, si, sj)
    return (oi, oj)


def kernel(user, item_i, item_j, embed_user, embed_item):
    return _run(user, item_i, item_j, embed_user, embed_item)


# speedup vs baseline: 3.4816x; 1.0009x over previous
import functools

import jax
import jax.numpy as jnp
from jax import lax
from jax.experimental import pallas as pl
from jax.experimental.pallas import tpu as pltpu
from jax.experimental.pallas import tpu_sc as plsc

BATCH = 16384
FACTOR = 64
LANES = 16
NUM_ROWS = 1000000
BLOCKS = (NUM_ROWS + 127) // 128
WBLK = 4
WCOLS = WBLK * 128
NWIN = 62
NBUCKET = NWIN * 3
HCAP = 4096

_info = plsc.get_sparse_core_info()
_NC, _NS = _info.num_cores, _info.num_subcores
_NW = _NC * _NS

_i32 = jnp.int32


def _splat(x):
    return jnp.full((LANES,), x, _i32)


def _sload(ref, i):
    v = ref[pl.ds((i >> 3) << 3, LANES)]
    g = v[_splat(i & 7)]
    return g[0]


def _sstore(ref, i, val):
    lane = lax.iota(_i32, LANES)
    plsc.store_scatter(ref, [_splat(i)], _splat(val), mask=lane == 0)


ICHUNK = 4096


def _sc_body(user_hbm, item_i_hbm, item_j_hbm, eu_hbm, ei_hbm,
             su_hbm, si_hbm, sj_hbm,
             idxA, idxB, hits, sorted_h, hist, base, cursor,
             colA, colB, colC, rowbuf,
             semA, semB, semC, semR):
    wid = lax.axis_index("s") * _NC + lax.axis_index("c")
    lane = lax.iota(_i32, LANES)
    lo = (wid * BLOCKS) >> 5
    hi = ((wid + 1) * BLOCKS) >> 5

    def wbase_of(w):
        uw = jnp.where(w < NWIN, w, w - NWIN)
        return jnp.minimum(lo + uw * WBLK, BLOCKS - WBLK)

    def fire(e, buf, sem):
        @pl.when(e < 2 * NWIN)
        def _f():
            r0 = wbase_of(e) * 128

            @pl.when(e < NWIN)
            def _u():
                pltpu.async_copy(
                    eu_hbm.at[pl.ds(0, FACTOR), pl.ds(r0, WCOLS)], buf, sem)

            @pl.when(e >= NWIN)
            def _i():
                pltpu.async_copy(
                    ei_hbm.at[pl.ds(0, FACTOR), pl.ds(r0, WCOLS)], buf, sem)

    fire(jnp.int32(0), colA, semA)
    fire(jnp.int32(1), colB, semB)

    idx_refs = (user_hbm, item_i_hbm, item_j_hbm)
    pieces = [(a, k) for a in range(3) for k in range(BATCH // ICHUNK)]
    ibufs = (idxA, idxB)
    isems = (semC, semR)

    def ifire(p):
        a, k = pieces[p]
        return pltpu.async_copy(
            idx_refs[a].at[pl.ds(k * ICHUNK, ICHUNK)], ibufs[p % 2],
            isems[p % 2])

    def scan_piece(buf, arr_id, k):
        def body(i, off):
            v = buf[pl.ds(i * LANES, LANES)]
            blk = v >> 7
            m = (blk >= lo) & (blk < hi)
            pc = plsc.all_reduce_population_count(m)

            @pl.when(pc[0] > 0)
            def _hit():
                local = blk - lo
                rloc = v & 127
                pos = k * ICHUNK + i * LANES + lane
                code = (local << 23) | (rloc << 16) | (arr_id << 14) | pos
                cum = plsc.cumsum(jnp.where(m, 1, 0))
                plsc.store_scatter(hits, [off + cum - 1], code, mask=m)

            return off + pc
        return body

    off = jnp.zeros((LANES,), _i32)
    cps = {0: ifire(0)}
    for p in range(len(pieces)):
        if p + 1 < len(pieces):
            cps[p + 1] = ifire(p + 1)
        cps.pop(p).wait()
        a, k = pieces[p]
        off = lax.fori_loop(0, ICHUNK // LANES, scan_piece(ibufs[p % 2], a, k),
                            off)
    n = off[0]
    sentinel = (NWIN * WBLK) << 23
    plsc.store_scatter(hits, [off + lane], _splat(sentinel),
                       mask=jnp.ones((LANES,), jnp.bool_))
    nch = (n + LANES - 1) // LANES

    def zero_hist(i, _):
        hist[pl.ds(i * LANES, LANES)] = jnp.zeros((LANES,), _i32)
        return _
    lax.fori_loop(0, (NBUCKET + LANES) // LANES, zero_hist, None)

    lane0 = lane == 0

    def histo(t, _):
        hv = hits[pl.ds(t * LANES, LANES)]
        bv = ((hv >> 23) >> 2) * 3 + ((hv >> 14) & 3)
        for k in range(LANES):
            bk = bv[_splat(k)]
            c = plsc.load_gather(hist, [bk])
            plsc.store_scatter(hist, [bk], c + 1, mask=lane0)
        return _
    lax.fori_loop(0, nch, histo, None)

    def prefix(i, carry):
        v = hist[pl.ds(i * LANES, LANES)]
        cum = plsc.cumsum(v)
        b = carry + cum - v
        base[pl.ds(i * LANES, LANES)] = b
        cursor[pl.ds(i * LANES, LANES)] = b
        return carry + _splat(lax.reduce_sum(v, (0,)))
    lax.fori_loop(0, (NBUCKET + LANES) // LANES, prefix,
                  jnp.zeros((LANES,), _i32))

    def csort(t, _):
        hv = hits[pl.ds(t * LANES, LANES)]
        bv = ((hv >> 23) >> 2) * 3 + ((hv >> 14) & 3)
        for k in range(LANES):
            bk = bv[_splat(k)]
            o = plsc.load_gather(cursor, [bk])
            plsc.store_scatter(cursor, [bk], o + 1, mask=lane0)
            plsc.store_scatter(sorted_h, [o], hv[_splat(k)], mask=lane0)
        return _
    lax.fori_loop(0, nch, csort, None)

    def do_bucket(b, shift, stage_hbm, buf, slot):
        bs = _sload(base, b)
        cnt = _sload(hist, b)

        def hit(t, slot):
            code = _sload(sorted_h, bs + t)
            col = shift + (((code >> 23) & 3) << 7) + ((code >> 16) & 127)
            pos = code & 16383
            ring = slot & 15

            @pl.when(slot >= 16)
            def _w():
                pltpu.make_async_copy(
                    rowbuf.at[0], stage_hbm.at[0], semR).wait()

            for c0 in range(0, FACTOR, LANES):
                v = plsc.load_gather(buf, [c0 + lane, _splat(col)])
                rowbuf[ring, pl.ds(c0, LANES)] = v
            pltpu.async_copy(rowbuf.at[ring], stage_hbm.at[pos], semR)
            return slot + 1

        return lax.fori_loop(0, cnt, hit, slot)

    def process(e, buf, sem, slot):
        def live(slot):
            pltpu.make_async_copy(
                eu_hbm.at[pl.ds(0, FACTOR), pl.ds(0, WCOLS)], buf, sem).wait()
            uw = jnp.where(e < NWIN, e, e - NWIN)
            shift = ((lo + uw * WBLK) - wbase_of(e)) * 128

            def if_user(slot):
                return do_bucket(uw * 3, shift, su_hbm, buf, slot)

            def if_item(slot):
                slot = do_bucket(uw * 3 + 1, shift, si_hbm, buf, slot)
                return do_bucket(uw * 3 + 2, shift, sj_hbm, buf, slot)

            return lax.cond(e < NWIN, if_user, if_item, slot)

        return lax.cond(e < 2 * NWIN, live, lambda s: s, slot)

    fire(jnp.int32(2), colC, semC)

    def sweep(p, slot):
        e0 = 3 * p
        slot = process(e0, colA, semA, slot)
        fire(e0 + 3, colA, semA)
        slot = process(e0 + 1, colB, semB, slot)
        fire(e0 + 4, colB, semB)
        slot = process(e0 + 2, colC, semC, slot)
        fire(e0 + 5, colC, semC)
        return slot

    slot = lax.fori_loop(0, (2 * NWIN + 2) // 3, sweep, jnp.int32(0))

    def drain(k, _):
        @pl.when(k < jnp.minimum(slot, 16))
        def _d():
            pltpu.make_async_copy(rowbuf.at[0], su_hbm.at[0], semR).wait()
        return _
    lax.fori_loop(0, 16, drain, None)


def _tc_body(su_ref, si_ref, sj_ref, oi_ref, oj_ref):
    u = su_ref[:, :FACTOR]
    vi = si_ref[:, :FACTOR]
    vj = sj_ref[:, :FACTOR]
    oi_ref[...] = jnp.sum(u * vi, axis=1)
    oj_ref[...] = jnp.sum(u * vj, axis=1)


@jax.jit
def _run(user, item_i, item_j, embed_user, embed_item):
    eu_t = embed_user.T
    ei_t = embed_item.T
    mesh = plsc.VectorSubcoreMesh(core_axis_name="c", subcore_axis_name="s")
    phase1 = functools.partial(
        pl.kernel,
        mesh=mesh,
        out_type=[
            jax.ShapeDtypeStruct((BATCH, 128), jnp.float32),
            jax.ShapeDtypeStruct((BATCH, 128), jnp.float32),
            jax.ShapeDtypeStruct((BATCH, 128), jnp.float32),
        ],
        scratch_types=[
            pltpu.VMEM((ICHUNK,), _i32),
            pltpu.VMEM((ICHUNK,), _i32),
            pltpu.VMEM((HCAP,), _i32),
            pltpu.VMEM((HCAP,), _i32),
            pltpu.VMEM((NBUCKET + LANES,), _i32),
            pltpu.VMEM((NBUCKET + LANES,), _i32),
            pltpu.VMEM((NBUCKET + LANES,), _i32),
            pltpu.VMEM((FACTOR, WCOLS), jnp.float32),
            pltpu.VMEM((FACTOR, WCOLS), jnp.float32),
            pltpu.VMEM((FACTOR, WCOLS), jnp.float32),
            pltpu.VMEM((16, 128), jnp.float32),
            pltpu.SemaphoreType.DMA,
            pltpu.SemaphoreType.DMA,
            pltpu.SemaphoreType.DMA,
            pltpu.SemaphoreType.DMA,
        ],
        compiler_params=pltpu.CompilerParams(
            needs_layout_passes=False, use_tc_tiling_on_sc=True
        ),
    )(_sc_body)
    su, si, sj = phase1(user, item_i, item_j, eu_t, ei_t)

    grid = 16
    rows = BATCH // grid
    oi, oj = pl.pallas_call(
        _tc_body,
        grid=(grid,),
        in_specs=[
            pl.BlockSpec((rows, 128), lambda i: (i, 0)),
            pl.BlockSpec((rows, 128), lambda i: (i, 0)),
            pl.BlockSpec((rows, 128), lambda i: (i, 0)),
        ],
        out_specs=[
            pl.BlockSpec((rows,), lambda i: (i,)),
            pl.BlockSpec((rows,), lambda i: (i,)),
        ],
        out_shape=[
            jax.ShapeDtypeStruct((BATCH,), jnp.float32),
            jax.ShapeDtypeStruct((BATCH,), jnp.float32),
        ],
    )(su, si, sj)
    return (oi, oj)


def kernel(user, item_i, item_j, embed_user, embed_item):
    return _run(user, item_i, item_j, embed_user, embed_item)
